# Initial kernel scaffold; baseline (speedup 1.0000x reference)
import jax
import jax.numpy as jnp
from jax.experimental import pallas as pl


def kernel(x, edge_index, W0, b0, g0, be0, W1, b1, g1, be1, W2, b2, g2, be2):
    raise NotImplementedError("write your pallas kernel here")


# speedup vs baseline: 2.7739x; 2.7739x over previous
import functools

import jax
import jax.numpy as jnp
from jax import lax
from jax.experimental import pallas as pl
from jax.experimental.pallas import tpu as pltpu
from jax.experimental.pallas import tpu_sc as plsc

N_NODES = 10000
N_EDGES = 320000
D = 128

NC = 2
NS = 16
NW = NC * NS
LANES = 16

H_PAD = 10240
NPT = H_PAD // NW
EPB = 128

CIN = 6400
NCH = N_EDGES // CIN
FLUSH = 2048
FCAP = FLUSH + 256
CAP = N_EDGES + 2560

_mesh = plsc.VectorSubcoreMesh(core_axis_name="c", subcore_axis_name="s")


def _wid():
    return lax.axis_index("s") * NC + lax.axis_index("c")


@functools.partial(
    pl.kernel,
    out_type=(
        jax.ShapeDtypeStruct((NW * CAP,), jnp.int32),
        jax.ShapeDtypeStruct((NW * CAP,), jnp.int32),
        jax.ShapeDtypeStruct((NW * 128,), jnp.int32),
    ),
    mesh=_mesh,
    compiler_params=pltpu.CompilerParams(needs_layout_passes=False),
    scratch_types=[
        pltpu.VMEM((CIN,), jnp.int32),
        pltpu.VMEM((CIN,), jnp.int32),
        pltpu.VMEM((FCAP,), jnp.int32),
        pltpu.VMEM((FCAP,), jnp.int32),
        pltpu.VMEM((128,), jnp.int32),
    ],
)
def _compact(src_hbm, dst_hbm, slist_hbm, dlist_hbm, cnt_hbm,
             sbuf, dbuf, lbs, lbd, cbuf):
    wid = _wid()
    lo = wid * NPT

    def chunk_body(c, carry):
        pltpu.sync_copy(src_hbm.at[pl.ds(pl.multiple_of(c * CIN, 128), CIN)], sbuf)
        pltpu.sync_copy(dst_hbm.at[pl.ds(pl.multiple_of(c * CIN, 128), CIN)], dbuf)

        def vec_body(i, carry2):
            off, goff = carry2
            sv = sbuf[pl.ds(i * LANES, LANES)]
            dv = dbuf[pl.ds(i * LANES, LANES)]
            dl = dv - lo
            m = (dl >= 0) & (dl < NPT)
            pc = jnp.sum(jnp.where(m, jnp.int32(1), jnp.int32(0)))
            plsc.store_compressed(lbs.at[pl.ds(off, LANES)], sv, mask=m)
            plsc.store_compressed(lbd.at[pl.ds(off, LANES)], dl, mask=m)
            off = off + pc
            flush = off >= FLUSH

            @pl.when(flush)
            def _():
                pltpu.sync_copy(lbs.at[pl.ds(0, FLUSH)],
                                slist_hbm.at[pl.ds(pl.multiple_of(wid * CAP + goff, 128), FLUSH)])
                pltpu.sync_copy(lbd.at[pl.ds(0, FLUSH)],
                                dlist_hbm.at[pl.ds(pl.multiple_of(wid * CAP + goff, 128), FLUSH)])
                lbs[pl.ds(0, LANES)] = lbs[pl.ds(FLUSH, LANES)]
                lbd[pl.ds(0, LANES)] = lbd[pl.ds(FLUSH, LANES)]

            off = jnp.where(flush, off - FLUSH, off)
            goff = jnp.where(flush, goff + FLUSH, goff)
            return off, goff

        return lax.fori_loop(0, CIN // LANES, vec_body, carry)

    off, goff = lax.fori_loop(
        0, NCH, chunk_body, (jnp.int32(0), jnp.int32(0)))

    zeros = jnp.zeros((LANES,), jnp.int32)
    trash = jnp.full((LANES,), NPT, jnp.int32)
    for k in range(EPB // LANES):
        lbs[pl.ds(off + k * LANES, LANES)] = zeros
        lbd[pl.ds(off + k * LANES, LANES)] = trash

    pltpu.sync_copy(lbs, slist_hbm.at[pl.ds(pl.multiple_of(wid * CAP + goff, 128), FCAP)])
    pltpu.sync_copy(lbd, dlist_hbm.at[pl.ds(pl.multiple_of(wid * CAP + goff, 128), FCAP)])

    total = goff + off
    nb = (total + EPB - 1) // EPB
    nbv = jnp.broadcast_to(nb, (16,))
    for k in range(8):
        cbuf[pl.ds(k * 16, 16)] = nbv
    pltpu.sync_copy(cbuf, cnt_hbm.at[pl.ds(pl.multiple_of(wid * 128, 128), 128)])


@functools.partial(
    pl.kernel,
    out_type=jax.ShapeDtypeStruct((H_PAD, D), jnp.float32),
    mesh=_mesh,
    compiler_params=pltpu.CompilerParams(needs_layout_passes=False),
    scratch_types=[
        pltpu.VMEM((NPT + 1, D), jnp.float32),
        pltpu.VMEM((EPB, D), jnp.float32),
        pltpu.VMEM((EPB,), jnp.int32),
        pltpu.VMEM((EPB,), jnp.int32),
        pltpu.VMEM((128,), jnp.int32),
        pltpu.SemaphoreType.DMA,
    ],
)
def _segmax(h_hbm, slist_hbm, dlist_hbm, cnt_hbm, agg_hbm,
            aggbuf, rows, sidx, dloc, cbuf, sem):
    wid = _wid()
    lo = wid * NPT
    pltpu.sync_copy(h_hbm.at[pl.ds(pl.multiple_of(lo, 8), NPT)], aggbuf.at[pl.ds(0, NPT)])
    pltpu.sync_copy(cnt_hbm.at[pl.ds(pl.multiple_of(wid * 128, 128), 128)], cbuf)
    nb = cbuf[pl.ds(0, LANES)][0]

    def batch_body(j, _):
        pltpu.sync_copy(slist_hbm.at[pl.ds(pl.multiple_of(wid * CAP + j * EPB, 128), EPB)], sidx)
        pltpu.sync_copy(dlist_hbm.at[pl.ds(pl.multiple_of(wid * CAP + j * EPB, 128), EPB)], dloc)
        pltpu.async_copy(h_hbm.at[sidx], rows, sem).wait()

        def group_body(g, _):
            dv = dloc[pl.ds(g * LANES, LANES)]
            for k in range(LANES):
                d = dv[k]
                e = g * LANES + k
                for c in range(D // LANES):
                    sl = pl.ds(c * LANES, LANES)
                    aggbuf[d, sl] = jnp.maximum(aggbuf[d, sl], rows[e, sl])
            return 0

        lax.fori_loop(0, EPB // LANES, group_body, 0)
        return 0

    lax.fori_loop(0, nb, batch_body, 0)
    pltpu.sync_copy(aggbuf.at[pl.ds(0, NPT)], agg_hbm.at[pl.ds(pl.multiple_of(lo, 8), NPT)])


BM = 256


def _dense_body(h_ref, a_ref, w_ref, b_ref, g_ref, be_ref, o_ref):
    t = h_ref[...] + a_ref[...]
    y = lax.dot_general(t, w_ref[...], (((1,), (1,)), ((), ())),
                        preferred_element_type=jnp.float32)
    y = y + b_ref[...]
    mu = jnp.mean(y, axis=1, keepdims=True)
    var = jnp.mean((y - mu) * (y - mu), axis=1, keepdims=True)
    y = (y - mu) * lax.rsqrt(var + 1e-5) * g_ref[...] + be_ref[...]
    o_ref[...] = jnp.where(y > 0, y, jnp.exp(y) - 1.0)


_dense = pl.pallas_call(
    _dense_body,
    out_shape=jax.ShapeDtypeStruct((H_PAD, D), jnp.float32),
    grid=(H_PAD // BM,),
    in_specs=[
        pl.BlockSpec((BM, D), lambda i: (i, 0)),
        pl.BlockSpec((BM, D), lambda i: (i, 0)),
        pl.BlockSpec((D, D), lambda i: (0, 0)),
        pl.BlockSpec((1, D), lambda i: (0, 0)),
        pl.BlockSpec((1, D), lambda i: (0, 0)),
        pl.BlockSpec((1, D), lambda i: (0, 0)),
    ],
    out_specs=pl.BlockSpec((BM, D), lambda i: (i, 0)),
)


def kernel(x, edge_index, W0, b0, g0, be0, W1, b1, g1, be1, W2, b2, g2, be2):
    h = jnp.pad(x, ((0, H_PAD - N_NODES), (0, 0)))
    src = edge_index[0]
    dst = edge_index[1]
    slist, dlist, cnt = _compact(src, dst)
    for (W, b, g, be) in ((W0, b0, g0, be0), (W1, b1, g1, be1),
                          (W2, b2, g2, be2)):
        agg = _segmax(h, slist, dlist, cnt)
        h = _dense(h, agg, W, b.reshape(1, D), g.reshape(1, D),
                   be.reshape(1, D))
    return h[:N_NODES]


# speedup vs baseline: 5.1609x; 1.8605x over previous
import functools

import jax
import jax.numpy as jnp
from jax import lax
from jax.experimental import pallas as pl
from jax.experimental.pallas import tpu as pltpu
from jax.experimental.pallas import tpu_sc as plsc

N_NODES = 10000
N_EDGES = 320000
D = 128

NC = 2
NS = 16
NW = NC * NS
LANES = 16

H_PAD = 10240
NPT = H_PAD // NW
EPB = 128
PADM = 2 * EPB

CIN = 3200
NCH = N_EDGES // CIN
VPC = CIN // LANES
SLIM = 16384
SL2 = SLIM + 512
CAP = N_EDGES + SL2 + 128
NBITS = 9

_mesh = plsc.VectorSubcoreMesh(core_axis_name="c", subcore_axis_name="s")
_sc_params = pltpu.CompilerParams(needs_layout_passes=False)


def _wid():
    return lax.axis_index("s") * NC + lax.axis_index("c")


@functools.partial(
    pl.kernel,
    out_type=(
        jax.ShapeDtypeStruct((NW * CAP,), jnp.int32),
        jax.ShapeDtypeStruct((NW * CAP,), jnp.int32),
        jax.ShapeDtypeStruct((NW * 128,), jnp.int32),
    ),
    mesh=_mesh,
    compiler_params=_sc_params,
    scratch_types=[
        pltpu.VMEM((CIN,), jnp.int32),
        pltpu.VMEM((CIN,), jnp.int32),
        pltpu.VMEM((CIN,), jnp.int32),
        pltpu.VMEM((CIN,), jnp.int32),
        pltpu.VMEM((SL2,), jnp.int32),
        pltpu.VMEM((SL2,), jnp.int32),
        pltpu.VMEM((SL2,), jnp.int32),
        pltpu.VMEM((SL2,), jnp.int32),
        pltpu.VMEM((128,), jnp.int32),
        pltpu.SemaphoreType.DMA,
        pltpu.SemaphoreType.DMA,
    ],
)
def _compact(src_hbm, dst_hbm, slist_hbm, dlist_hbm, cnt_hbm,
             sb0, db0, sb1, db1, lbs, lbd, pbs, pbd, cbuf, sem0, sem1):
    wid = _wid()
    lo = wid * NPT
    lbase = wid * CAP

    def fire(c, sb, db, sem):
        off_in = pl.multiple_of(c * CIN, 128)
        pltpu.async_copy(src_hbm.at[pl.ds(off_in, CIN)], sb, sem)
        pltpu.async_copy(dst_hbm.at[pl.ds(off_in, CIN)], db, sem)

    def drain(sb, db, sem):
        pltpu.make_async_copy(src_hbm.at[pl.ds(0, CIN)], sb, sem).wait()
        pltpu.make_async_copy(dst_hbm.at[pl.ds(0, CIN)], db, sem).wait()

    def process(sbuf, dbuf, carry):
        def grp_body(gi, carry2):
            off, goff = carry2
            for v in range(8):
                sl = pl.ds(gi * 8 * LANES + v * LANES, LANES)
                sv = sbuf[sl]
                dv = dbuf[sl]
                dl = dv - lo
                m = lax.bitcast_convert_type(dl, jnp.uint32) < jnp.uint32(NPT)
                pc = jnp.sum(jnp.where(m, jnp.int32(1), jnp.int32(0)))
                plsc.store_compressed(lbs.at[pl.ds(off, LANES)], sv, mask=m)
                plsc.store_compressed(lbd.at[pl.ds(off, LANES)], dl, mask=m)
                off = off + pc
            spill = off >= SLIM

            @pl.when(spill)
            def _():
                dst = pl.multiple_of(lbase + goff, 128)
                pltpu.sync_copy(lbs.at[pl.ds(0, SLIM)],
                                slist_hbm.at[pl.ds(dst, SLIM)])
                pltpu.sync_copy(lbd.at[pl.ds(0, SLIM)],
                                dlist_hbm.at[pl.ds(dst, SLIM)])
                for v in range(8):
                    tsl = pl.ds(v * LANES, LANES)
                    ssl = pl.ds(SLIM + v * LANES, LANES)
                    lbs[tsl] = lbs[ssl]
                    lbd[tsl] = lbd[ssl]

            off = jnp.where(spill, off - SLIM, off)
            goff = jnp.where(spill, goff + SLIM, goff)
            return off, goff

        return lax.fori_loop(0, VPC // 8, grp_body, carry)

    fire(0, sb0, db0, sem0)

    def pair_body(t, carry):
        fire(2 * t + 1, sb1, db1, sem1)
        drain(sb0, db0, sem0)
        carry = process(sb0, db0, carry)

        @pl.when(t < NCH // 2 - 1)
        def _():
            fire(2 * t + 2, sb0, db0, sem0)

        drain(sb1, db1, sem1)
        return process(sb1, db1, carry)

    off, goff = lax.fori_loop(
        0, NCH // 2, pair_body, (jnp.int32(0), jnp.int32(0)))

    zeros = jnp.zeros((LANES,), jnp.int32)
    trash = jnp.full((LANES,), NPT, jnp.int32)

    pad0 = (LANES - off % LANES) % LANES

    @pl.when(pad0 > 0)
    def _():
        lbs[pl.ds(off, LANES)] = zeros
        lbd[pl.ds(off, LANES)] = trash

    off16 = off + pad0
    total = goff + off16
    sorted_flag = goff == 0

    @pl.when(sorted_flag)
    def _():
        nvec = off16 // LANES

        def zc_body(i, z):
            dv = lbd[pl.ds(i * LANES, LANES)]
            b = dv & 1
            return z + jnp.sum(jnp.where(b == 0, jnp.int32(1), jnp.int32(0)))

        zc = lax.fori_loop(0, nvec, zc_body, jnp.int32(0))

        def make_pass(p, src_s, src_d, dst_s, dst_d):
            def pass_body(i, carry):
                c0, c1, zn = carry
                sv = src_s[pl.ds(i * LANES, LANES)]
                dv = src_d[pl.ds(i * LANES, LANES)]
                bit = (dv >> p) & 1
                m0 = bit == 0
                m1 = bit == 1
                pc0 = jnp.sum(jnp.where(m0, jnp.int32(1), jnp.int32(0)))
                plsc.store_compressed(dst_s.at[pl.ds(c0, LANES)], sv, mask=m0)
                plsc.store_compressed(dst_d.at[pl.ds(c0, LANES)], dv, mask=m0)
                plsc.store_compressed(dst_s.at[pl.ds(c1, LANES)], sv, mask=m1)
                plsc.store_compressed(dst_d.at[pl.ds(c1, LANES)], dv, mask=m1)
                bn = (dv >> (p + 1)) & 1
                zn = zn + jnp.sum(jnp.where(bn == 0, jnp.int32(1),
                                            jnp.int32(0)))
                return c0 + pc0, c1 + (LANES - pc0), zn

            return pass_body

        z = zc
        for p in range(NBITS):
            if p % 2 == 0:
                body = make_pass(p, lbs, lbd, pbs, pbd)
            else:
                body = make_pass(p, pbs, pbd, lbs, lbd)
            _, _, z = lax.fori_loop(0, nvec, body,
                                    (jnp.int32(0), z, jnp.int32(0)))

        def copy_back(i, _):
            lbs[pl.ds(i * LANES, LANES)] = pbs[pl.ds(i * LANES, LANES)]
            lbd[pl.ds(i * LANES, LANES)] = pbd[pl.ds(i * LANES, LANES)]
            return 0

        lax.fori_loop(0, nvec, copy_back, 0)

    target = jnp.maximum(((total + PADM - 1) // PADM) * PADM, PADM)
    n_dummy_vec = (target - total) // LANES

    def dummy_body(k, off2):
        lbs[pl.ds(off2, LANES)] = zeros
        lbd[pl.ds(off2, LANES)] = trash
        return off2 + LANES

    lax.fori_loop(0, n_dummy_vec, dummy_body, off16)

    dst = pl.multiple_of(lbase + goff, 128)
    pltpu.sync_copy(lbs, slist_hbm.at[pl.ds(dst, SL2)])
    pltpu.sync_copy(lbd, dlist_hbm.at[pl.ds(dst, SL2)])

    nb = target // EPB
    nbv = jnp.broadcast_to(nb, (16,))
    for k in range(8):
        cbuf[pl.ds(k * 16, 16)] = nbv
    pltpu.sync_copy(cbuf, cnt_hbm.at[pl.ds(pl.multiple_of(wid * 128, 128), 128)])


@functools.partial(
    pl.kernel,
    out_type=jax.ShapeDtypeStruct((H_PAD, D), jnp.float32),
    mesh=_mesh,
    compiler_params=_sc_params,
    scratch_types=[
        pltpu.VMEM((NPT + 1, D), jnp.float32),
        pltpu.VMEM((EPB, D), jnp.float32),
        pltpu.VMEM((EPB, D), jnp.float32),
        pltpu.VMEM((EPB,), jnp.int32),
        pltpu.VMEM((EPB,), jnp.int32),
        pltpu.VMEM((EPB,), jnp.int32),
        pltpu.VMEM((EPB,), jnp.int32),
        pltpu.VMEM((128,), jnp.int32),
        pltpu.SemaphoreType.DMA,
        pltpu.SemaphoreType.DMA,
        pltpu.SemaphoreType.DMA,
        pltpu.SemaphoreType.DMA,
        pltpu.SemaphoreType.DMA,
        pltpu.SemaphoreType.DMA,
    ],
)
def _segmax(h_hbm, slist_hbm, dlist_hbm, cnt_hbm, agg_hbm,
            aggbuf, rows0, rows1, bs0, bs1, bd0, bd1, cbuf,
            semg0, semg1, sems0, sems1, semd0, semd1):
    wid = _wid()
    lo = wid * NPT
    lbase = wid * CAP
    pltpu.sync_copy(h_hbm.at[pl.ds(pl.multiple_of(lo, 8), NPT)],
                    aggbuf.at[pl.ds(0, NPT)])
    pltpu.sync_copy(cnt_hbm.at[pl.ds(pl.multiple_of(wid * 128, 128), 128)],
                    cbuf)
    nb = cbuf[pl.ds(0, LANES)][0]

    rows = (rows0, rows1)
    bs = (bs0, bs1)
    bd = (bd0, bd1)
    semg = (semg0, semg1)
    sems = (sems0, sems1)
    semd = (semd0, semd1)

    def fire_bs(j, p):
        off = pl.multiple_of(lbase + j * EPB, 128)
        pltpu.async_copy(slist_hbm.at[pl.ds(off, EPB)], bs[p], sems[p])

    def fire_bd(j, p):
        off = pl.multiple_of(lbase + j * EPB, 128)
        pltpu.async_copy(dlist_hbm.at[pl.ds(off, EPB)], bd[p], semd[p])

    def drain_bs(p):
        pltpu.make_async_copy(slist_hbm.at[pl.ds(0, EPB)], bs[p],
                              sems[p]).wait()

    def drain_bd(p):
        pltpu.make_async_copy(dlist_hbm.at[pl.ds(0, EPB)], bd[p],
                              semd[p]).wait()

    def fire_g(p):
        pltpu.async_copy(h_hbm.at[bs[p]], rows[p], semg[p])

    def drain_g(p):
        pltpu.make_async_copy(h_hbm.at[pl.ds(0, EPB)], rows[p],
                              semg[p]).wait()

    def process(p, carry):
        def grp_body(g, carry2):
            dv = bd[p][pl.ds(g * LANES, LANES)]
            for k in range(LANES):
                cur_d = carry2[0]
                accs = carry2[1:]
                d = dv[k]

                def flush_and_load(args):
                    cd, a = args
                    for c in range(D // LANES):
                        aggbuf[cd, pl.ds(c * LANES, LANES)] = a[c]
                    return tuple(aggbuf[d, pl.ds(c * LANES, LANES)]
                                 for c in range(D // LANES))

                def keep(args):
                    return args[1]

                accs = lax.cond(d != cur_d, flush_and_load, keep,
                                (cur_d, accs))
                e = g * LANES + k
                accs = tuple(
                    jnp.maximum(accs[c], rows[p][e, pl.ds(c * LANES, LANES)])
                    for c in range(D // LANES))
                carry2 = (d,) + accs
            return carry2

        return lax.fori_loop(0, EPB // LANES, grp_body, carry)

    def turn(j, p, carry):
        @pl.when(j + 1 < nb)
        def _():
            drain_bs(1 - p)
            fire_g(1 - p)

        drain_g(p)

        @pl.when(j + 2 < nb)
        def _():
            fire_bs(j + 2, p)

        drain_bd(p)
        carry = process(p, carry)

        @pl.when(j + 2 < nb)
        def _():
            fire_bd(j + 2, p)

        return carry

    fire_bs(0, 0)
    fire_bd(0, 0)
    fire_bs(1, 1)
    fire_bd(1, 1)
    drain_bs(0)
    fire_g(0)

    init = (jnp.int32(NPT),) + tuple(
        aggbuf[NPT, pl.ds(c * LANES, LANES)] for c in range(D // LANES))

    def pair_body(t, carry):
        carry = turn(2 * t, 0, carry)
        return turn(2 * t + 1, 1, carry)

    carry = lax.fori_loop(0, nb // 2, pair_body, init)
    cur_d = carry[0]
    for c in range(D // LANES):
        aggbuf[cur_d, pl.ds(c * LANES, LANES)] = carry[1 + c]

    pltpu.sync_copy(aggbuf.at[pl.ds(0, NPT)],
                    agg_hbm.at[pl.ds(pl.multiple_of(lo, 8), NPT)])


BM = 256


def _dense_body(h_ref, a_ref, w_ref, b_ref, g_ref, be_ref, o_ref):
    t = h_ref[...] + a_ref[...]
    y = lax.dot_general(t, w_ref[...], (((1,), (1,)), ((), ())),
                        preferred_element_type=jnp.float32)
    y = y + b_ref[...]
    mu = jnp.mean(y, axis=1, keepdims=True)
    var = jnp.mean((y - mu) * (y - mu), axis=1, keepdims=True)
    y = (y - mu) * lax.rsqrt(var + 1e-5) * g_ref[...] + be_ref[...]
    o_ref[...] = jnp.where(y > 0, y, jnp.exp(y) - 1.0)


_dense = pl.pallas_call(
    _dense_body,
    out_shape=jax.ShapeDtypeStruct((H_PAD, D), jnp.float32),
    grid=(H_PAD // BM,),
    in_specs=[
        pl.BlockSpec((BM, D), lambda i: (i, 0)),
        pl.BlockSpec((BM, D), lambda i: (i, 0)),
        pl.BlockSpec((D, D), lambda i: (0, 0)),
        pl.BlockSpec((1, D), lambda i: (0, 0)),
        pl.BlockSpec((1, D), lambda i: (0, 0)),
        pl.BlockSpec((1, D), lambda i: (0, 0)),
    ],
    out_specs=pl.BlockSpec((BM, D), lambda i: (i, 0)),
)


def kernel(x, edge_index, W0, b0, g0, be0, W1, b1, g1, be1, W2, b2, g2, be2):
    h = jnp.pad(x, ((0, H_PAD - N_NODES), (0, 0)))
    src = edge_index[0]
    dst = edge_index[1]
    slist, dlist, cnt = _compact(src, dst)
    for (W, b, g, be) in ((W0, b0, g0, be0), (W1, b1, g1, be1),
                          (W2, b2, g2, be2)):
        agg = _segmax(h, slist, dlist, cnt)
        h = _dense(h, agg, W, b.reshape(1, D), g.reshape(1, D),
                   be.reshape(1, D))
    return h[:N_NODES]
